# Initial kernel scaffold; baseline (speedup 1.0000x reference)
import jax
import jax.numpy as jnp
from jax.experimental import pallas as pl


def kernel(x, edge_index, diff_edge_index, diff_edge_weight, shuf_x, W1, b1, W2, b2, prelu_a, Wb, bb):
    raise NotImplementedError("write your pallas kernel here")


# speedup vs baseline: 8.5011x; 8.5011x over previous
import functools

import jax
import jax.numpy as jnp
from jax import lax
from jax.experimental import pallas as pl
from jax.experimental.pallas import tpu as pltpu
from jax.experimental.pallas import tpu_sc as plsc

N = 10000
NP = 10240
E = 320000
D = 128
NC = 2
NS = 16
CH = 80

EPT = E // NS
NCH = EPT // CH
SB = 50
NB = NCH // SB
STR = NP // NS

DPT = E // (NC * NS)
DNCH = DPT // CH
DW = 16
ZR = 16

_mesh = plsc.VectorSubcoreMesh(core_axis_name="c", subcore_axis_name="s")


@functools.partial(
    pl.kernel,
    out_type=jax.ShapeDtypeStruct((NC * NP, DW), jnp.float32),
    mesh=_mesh,
    scratch_types=[
        pltpu.VMEM((DNCH, CH), jnp.int32),
        pltpu.VMEM((CH, DW), jnp.float32),
        pltpu.VMEM((ZR, DW), jnp.float32),
        pltpu.VMEM_SHARED((NP, DW), jnp.float32),
    ],
)
def _deg_sc(dst_h, out_h, idx_v, ones_v, zb_v, acc_s):
    c = lax.axis_index("c")
    s = lax.axis_index("s")
    wid = c * NS + s

    def fill(e, _):
        ones_v[e, :] = jnp.full((DW,), 1.0, jnp.float32)
        return 0

    lax.fori_loop(0, CH, fill, 0)

    def zfill(e, _):
        zb_v[e, :] = jnp.full((DW,), 0.0, jnp.float32)
        return 0

    lax.fori_loop(0, ZR, zfill, 0)

    r0 = s * STR
    for q in range(STR // ZR):
        pltpu.sync_copy(zb_v, acc_s.at[pl.ds(r0 + q * ZR, ZR)])
    plsc.subcore_barrier()

    pltpu.sync_copy(dst_h.at[wid], idx_v)

    def chunk(k, _):
        pltpu.sync_copy(ones_v, acc_s.at[idx_v.at[k]], add=True)
        return 0

    lax.fori_loop(0, DNCH, chunk, 0)
    plsc.subcore_barrier()
    pltpu.sync_copy(acc_s.at[pl.ds(r0, STR)], out_h.at[pl.ds(c * NP + r0, STR)])


def _prep_body(x_ref, sx_ref, w1_ref, w2_ref, degp_ref, t1_ref, t2_ref):
    deg = degp_ref[0, :, 0:1] + degp_ref[1, :, 0:1] + 1.0
    dinv = lax.rsqrt(deg)
    xb = x_ref[...]
    sb = sx_ref[...]
    w1 = w1_ref[...]
    w2 = w2_ref[...]
    t1_ref[0] = jnp.dot(xb, w1, preferred_element_type=jnp.float32) * dinv
    t1_ref[1] = jnp.dot(sb, w1, preferred_element_type=jnp.float32) * dinv
    t2_ref[0] = jnp.dot(xb, w2, preferred_element_type=jnp.float32)
    t2_ref[1] = jnp.dot(sb, w2, preferred_element_type=jnp.float32)


_BR = 1000


def _prep_call(x, sx, w1, w2, degp):
    return pl.pallas_call(
        _prep_body,
        grid=(N // _BR,),
        in_specs=[
            pl.BlockSpec((_BR, D), lambda i: (i, 0)),
            pl.BlockSpec((_BR, D), lambda i: (i, 0)),
            pl.BlockSpec((D, D), lambda i: (0, 0)),
            pl.BlockSpec((D, D), lambda i: (0, 0)),
            pl.BlockSpec((NC, _BR, DW), lambda i: (0, i, 0)),
        ],
        out_specs=[
            pl.BlockSpec((NC, _BR, D), lambda i: (0, i, 0)),
            pl.BlockSpec((NC, _BR, D), lambda i: (0, i, 0)),
        ],
        out_shape=[
            jax.ShapeDtypeStruct((NC, N, D), jnp.float32),
            jax.ShapeDtypeStruct((NC, N, D), jnp.float32),
        ],
    )(x, sx, w1, w2, degp)


@functools.partial(
    pl.kernel,
    out_type=[
        jax.ShapeDtypeStruct((NC * NP, D), jnp.float32),
        jax.ShapeDtypeStruct((NC * NP, D), jnp.float32),
    ],
    mesh=_mesh,
    scratch_types=[
        pltpu.VMEM((SB, CH), jnp.int32),
        pltpu.VMEM((SB, CH), jnp.int32),
        pltpu.VMEM((CH, 16), jnp.float32),
        pltpu.VMEM((CH, D), jnp.float32),
        pltpu.VMEM_SHARED((NP, D), jnp.float32),
    ],
)
def _agg_sc(t1_h, t2_h, src1_h, dst1_h, src2_h, dst2_h, w_h,
            s1_h, s2_h, isrc_v, idst_v, w_v, rows_v, acc_s):
    c = lax.axis_index("c")
    s = lax.axis_index("s")
    r0 = s * STR
    coff = c * NP

    def zero_stripe():
        def zfill(e, _):
            for j in range(D // 16):
                rows_v[e, pl.ds(j * 16, 16)] = jnp.full((16,), 0.0,
                                                        jnp.float32)
            return 0

        lax.fori_loop(0, CH, zfill, 0)
        for q in range(STR // CH):
            pltpu.sync_copy(rows_v, acc_s.at[pl.ds(r0 + q * CH, CH)])

    def edge_pass(src_h, dst_h, tbl_h, weighted):
        def block(b, _):
            pltpu.sync_copy(src_h.at[s, b], isrc_v)
            pltpu.sync_copy(dst_h.at[s, b], idst_v)

            def adj(k, _):
                for j in range(CH // 16):
                    sl = pl.ds(j * 16, 16)
                    isrc_v[k, sl] = isrc_v[k, sl] + coff
                return 0

            lax.fori_loop(0, SB, adj, 0)

            def chunk(k, _):
                pltpu.sync_copy(tbl_h.at[isrc_v.at[k]], rows_v)
                if weighted:
                    pltpu.sync_copy(w_h.at[s, b * SB + k], w_v)

                    def mul_e(e, _):
                        wf = w_v[e, :]
                        for j in range(D // 16):
                            sl = pl.ds(j * 16, 16)
                            rows_v[e, sl] = rows_v[e, sl] * wf
                        return 0

                    lax.fori_loop(0, CH, mul_e, 0)
                pltpu.sync_copy(rows_v, acc_s.at[idst_v.at[k]], add=True)
                return 0

            lax.fori_loop(0, SB, chunk, 0)
            return 0

        lax.fori_loop(0, NB, block, 0)

    def stripe_out(out_h):
        pltpu.sync_copy(acc_s.at[pl.ds(r0, STR)],
                        out_h.at[pl.ds(coff + r0, STR)])

    zero_stripe()
    plsc.subcore_barrier()
    edge_pass(src1_h, dst1_h, t1_h, False)
    plsc.subcore_barrier()
    stripe_out(s1_h)
    zero_stripe()
    plsc.subcore_barrier()
    edge_pass(src2_h, dst2_h, t2_h, True)
    plsc.subcore_barrier()
    stripe_out(s2_h)


def _read_body(s1_ref, s2_ref, t1_ref, degp_ref, b1_ref, b2_ref, pa_ref,
               wb_ref, bb_ref, o_ref):
    deg = degp_ref[0, :, 0:1] + degp_ref[1, :, 0:1] + 1.0
    dinv = lax.rsqrt(deg)
    a = pa_ref[0, 0]

    def pr(z):
        return jnp.where(z > 0, z, a * z)

    h1 = pr((s1_ref[0] + t1_ref[0]) * dinv + b1_ref[...])
    h3 = pr((s1_ref[1] + t1_ref[1]) * dinv + b1_ref[...])
    h2 = pr(s2_ref[0] + b2_ref[...])
    h4 = pr(s2_ref[1] + b2_ref[...])
    c1 = jax.nn.sigmoid(jnp.mean(h1, axis=1, keepdims=True))
    c2 = jax.nn.sigmoid(jnp.mean(h2, axis=1, keepdims=True))
    u = jnp.sum(wb_ref[...], axis=1, keepdims=True)
    bbs = bb_ref[0, 0]
    q1 = jnp.dot(h1, u, preferred_element_type=jnp.float32)
    q2 = jnp.dot(h2, u, preferred_element_type=jnp.float32)
    q3 = jnp.dot(h3, u, preferred_element_type=jnp.float32)
    q4 = jnp.dot(h4, u, preferred_element_type=jnp.float32)
    o_ref[:, 0:1] = c1 * q2 + bbs
    o_ref[:, 1:2] = c2 * q1 + bbs
    o_ref[:, 2:3] = c1 * q4 + bbs
    o_ref[:, 3:4] = c2 * q3 + bbs


def _read_call(s1, s2, t1, degp, b1, b2, pa, wb, bb):
    return pl.pallas_call(
        _read_body,
        grid=(N // _BR,),
        in_specs=[
            pl.BlockSpec((NC, _BR, D), lambda i: (0, i, 0)),
            pl.BlockSpec((NC, _BR, D), lambda i: (0, i, 0)),
            pl.BlockSpec((NC, _BR, D), lambda i: (0, i, 0)),
            pl.BlockSpec((NC, _BR, DW), lambda i: (0, i, 0)),
            pl.BlockSpec((1, D), lambda i: (0, 0)),
            pl.BlockSpec((1, D), lambda i: (0, 0)),
            pl.BlockSpec((1, 1), lambda i: (0, 0)),
            pl.BlockSpec((D, D), lambda i: (0, 0)),
            pl.BlockSpec((1, 1), lambda i: (0, 0)),
        ],
        out_specs=pl.BlockSpec((_BR, 4), lambda i: (i, 0)),
        out_shape=jax.ShapeDtypeStruct((N, 4), jnp.float32),
    )(s1, s2, t1, degp, b1, b2, pa, wb, bb)


def kernel(x, edge_index, diff_edge_index, diff_edge_weight, shuf_x,
           W1, b1, W2, b2, prelu_a, Wb, bb):
    src1 = edge_index[0].astype(jnp.int32).reshape(NS, NB, SB, CH)
    dst1 = edge_index[1].astype(jnp.int32).reshape(NS, NB, SB, CH)
    src2 = diff_edge_index[0].astype(jnp.int32).reshape(NS, NB, SB, CH)
    dst2 = diff_edge_index[1].astype(jnp.int32).reshape(NS, NB, SB, CH)
    wv = jnp.broadcast_to(
        diff_edge_weight.astype(jnp.float32).reshape(E, 1),
        (E, 16)).reshape(NS, NCH, CH, 16)
    dstd = edge_index[1].astype(jnp.int32).reshape(NC * NS, DNCH, CH)

    degp = _deg_sc(dstd).reshape(NC, NP, DW)[:, :N]
    t1, t2 = _prep_call(x, shuf_x, W1, W2, degp)
    pad = ((0, 0), (0, NP - N), (0, 0))
    s1, s2 = _agg_sc(jnp.pad(t1, pad).reshape(NC * NP, D),
                     jnp.pad(t2, pad).reshape(NC * NP, D),
                     src1, dst1, src2, dst2, wv)
    out = _read_call(s1.reshape(NC, NP, D)[:, :N],
                     s2.reshape(NC, NP, D)[:, :N], t1, degp,
                     b1.reshape(1, D), b2.reshape(1, D),
                     prelu_a.reshape(1, 1), Wb.reshape(D, D),
                     bb.reshape(1, 1))
    return jnp.transpose(out).reshape(4 * N)
